# Initial kernel scaffold; baseline (speedup 1.0000x reference)
import jax
import jax.numpy as jnp
from jax.experimental import pallas as pl


def kernel(features, edge_index, conv_w, conv_b, lin1_w, lin1_b, lin2_w, lin2_b, s1_self, s1_neigh, s1_b, s2_self, s2_neigh, s2_b):
    raise NotImplementedError("write your pallas kernel here")


# speedup vs baseline: 3.3398x; 3.3398x over previous
import functools

import jax
import jax.numpy as jnp
from jax import lax
from jax.experimental import pallas as pl
from jax.experimental.pallas import tpu as pltpu
from jax.experimental.pallas import tpu_sc as plsc

N = 10000
E = 640000
NCLS = 16

BN = 64
NP = 10240
NT = NP + 64
NW = 32
CH = 128
KCH = 160
EP = NW * KCH * CH


def _cnn_block(f_ref, bcat_ref, brow_ref, w1_ref, b1_ref, w2_ref, b2_ref,
               out_ref):
    F = f_ref[...]
    rows = BN * 20
    zrow = jnp.zeros((1, 64), jnp.float32)
    g0 = jnp.concatenate([zrow, F[:-1, :]], axis=0)
    g2 = jnp.concatenate([F[1:, :], zrow], axis=0)
    h_idx = jax.lax.broadcasted_iota(jnp.int32, (rows, 1), 0) % 20
    g0 = jnp.where(h_idx == 0, 0.0, g0)
    g2 = jnp.where(h_idx == 19, 0.0, g2)
    gcat = jnp.concatenate([g0, F, g2], axis=1)
    y = jnp.dot(gcat, bcat_ref[...], preferred_element_type=jnp.float32)
    y = jnp.maximum(y + brow_ref[...], 0.0)
    y4 = y.reshape(BN * 10, 2, 2048)
    yh = jnp.maximum(y4[:, 0, :], y4[:, 1, :])
    yw = jnp.maximum(yh, pltpu.roll(yh, shift=2047, axis=1))
    y3 = yw.reshape(BN, 10, 2048)
    acc = jnp.broadcast_to(b1_ref[...], (BN, 32))
    for ph in range(10):
        acc = acc + jnp.dot(y3[:, ph, :], w1_ref[ph],
                            preferred_element_type=jnp.float32)
    z = jnp.maximum(acc, 0.0)
    x = jnp.dot(z, w2_ref[...], preferred_element_type=jnp.float32)
    out_ref[...] = jnp.maximum(x + b2_ref[...], 0.0)


def _cnn(feats2d, bcat, brow, w1p, b1r, w2p, b2r):
    return pl.pallas_call(
        _cnn_block,
        grid=(NP // BN,),
        in_specs=[
            pl.BlockSpec((BN * 20, 64), lambda i: (i, 0)),
            pl.BlockSpec((192, 2048), lambda i: (0, 0)),
            pl.BlockSpec((1, 2048), lambda i: (0, 0)),
            pl.BlockSpec((10, 2048, 32), lambda i: (0, 0, 0)),
            pl.BlockSpec((1, 32), lambda i: (0, 0)),
            pl.BlockSpec((32, 32), lambda i: (0, 0)),
            pl.BlockSpec((1, 32), lambda i: (0, 0)),
        ],
        out_specs=pl.BlockSpec((BN, 32), lambda i: (i, 0)),
        out_shape=jax.ShapeDtypeStruct((NP, 32), jnp.float32),
    )(feats2d, bcat, brow, w1p, b1r, w2p, b2r)


def _make_scatter(D, with_deg):
    mesh = plsc.VectorSubcoreMesh(core_axis_name="c", subcore_axis_name="s")
    out_type = [jax.ShapeDtypeStruct((2, NT, D), jnp.float32)]
    scratch = [
        pltpu.VMEM((KCH, CH), jnp.int32),
        pltpu.VMEM((KCH, CH), jnp.int32),
        pltpu.VMEM((CH, D), jnp.float32),
        pltpu.VMEM_SHARED((NT, D), jnp.float32),
        pltpu.SemaphoreType.DMA,
    ]
    if with_deg:
        out_type.append(jax.ShapeDtypeStruct((2, NT, 8), jnp.float32))
        scratch += [
            pltpu.VMEM((CH, 8), jnp.float32),
            pltpu.VMEM_SHARED((NT, 8), jnp.float32),
        ]

    def body(*refs):
        if with_deg:
            (x_hbm, src_hbm, dst_hbm, z_hbm, z8_hbm, ones_hbm,
             agg_out, deg_out, src_v, dst_v, msg_v, agg_sh, sem,
             ones_v, deg_sh) = refs
        else:
            (x_hbm, src_hbm, dst_hbm, z_hbm,
             agg_out, src_v, dst_v, msg_v, agg_sh, sem) = refs
        c = lax.axis_index("c")
        s = lax.axis_index("s")
        wid = s * 2 + c
        pltpu.sync_copy(src_hbm.at[wid], src_v)
        pltpu.sync_copy(dst_hbm.at[wid], dst_v)
        if with_deg:
            pltpu.sync_copy(ones_hbm, ones_v)

        @pl.when(s == 0)
        def _init():
            pltpu.sync_copy(z_hbm, agg_sh)
            if with_deg:
                pltpu.sync_copy(z8_hbm, deg_sh)

        plsc.subcore_barrier()

        def step(j, carry):
            pltpu.async_copy(x_hbm.at[src_v.at[j]], msg_v, sem).wait()
            pltpu.sync_copy(msg_v, agg_sh.at[dst_v.at[j]], add=True)
            if with_deg:
                pltpu.sync_copy(ones_v, deg_sh.at[dst_v.at[j]], add=True)
            return carry

        lax.fori_loop(0, KCH, step, 0)
        plsc.subcore_barrier()

        @pl.when(s == 0)
        def _flush():
            pltpu.sync_copy(agg_sh, agg_out.at[c])
            if with_deg:
                pltpu.sync_copy(deg_sh, deg_out.at[c])

    return pl.kernel(
        body,
        out_type=tuple(out_type) if with_deg else out_type[0],
        mesh=mesh,
        scratch_types=scratch,
        compiler_params=pltpu.CompilerParams(use_tc_tiling_on_sc=False),
    )


def _combine1_block(x_ref, a_ref, d_ref, s1s_ref, s1n_ref, s1b_ref,
                    s2n_ref, s2s_ref, s2b_ref, p_ref, self2_ref):
    agg = a_ref[0] + a_ref[1]
    deg = d_ref[0, :, 0:1] + d_ref[1, :, 0:1]
    rdeg = 1.0 / jnp.maximum(deg, 1.0)
    h = jnp.dot(x_ref[...], s1s_ref[...], preferred_element_type=jnp.float32)
    h = h + jnp.dot(agg * rdeg, s1n_ref[...],
                    preferred_element_type=jnp.float32)
    h = jnp.maximum(h + s1b_ref[...], 0.0)
    p_ref[...] = jnp.dot(h, s2n_ref[...], preferred_element_type=jnp.float32)
    self2_ref[...] = (jnp.dot(h, s2s_ref[...],
                              preferred_element_type=jnp.float32)
                      + s2b_ref[...])


def _combine1(x, aggp, degp, s1sT, s1nT, s1b, s2nT, s2sT, s2b):
    B = 1280
    return pl.pallas_call(
        _combine1_block,
        grid=(NP // B,),
        in_specs=[
            pl.BlockSpec((B, 32), lambda i: (i, 0)),
            pl.BlockSpec((2, B, 32), lambda i: (0, i, 0)),
            pl.BlockSpec((2, B, 8), lambda i: (0, i, 0)),
            pl.BlockSpec((32, 64), lambda i: (0, 0)),
            pl.BlockSpec((32, 64), lambda i: (0, 0)),
            pl.BlockSpec((1, 64), lambda i: (0, 0)),
            pl.BlockSpec((64, NCLS), lambda i: (0, 0)),
            pl.BlockSpec((64, NCLS), lambda i: (0, 0)),
            pl.BlockSpec((1, NCLS), lambda i: (0, 0)),
        ],
        out_specs=[
            pl.BlockSpec((B, NCLS), lambda i: (i, 0)),
            pl.BlockSpec((B, NCLS), lambda i: (i, 0)),
        ],
        out_shape=[
            jax.ShapeDtypeStruct((NP, NCLS), jnp.float32),
            jax.ShapeDtypeStruct((NP, NCLS), jnp.float32),
        ],
    )(x, aggp, degp, s1sT, s1nT, s1b, s2nT, s2sT, s2b)


def _combine2_block(self2_ref, a_ref, d_ref, out_ref):
    agg = a_ref[0] + a_ref[1]
    deg = d_ref[0, :, 0:1] + d_ref[1, :, 0:1]
    rdeg = 1.0 / jnp.maximum(deg, 1.0)
    out_ref[...] = self2_ref[...] + agg * rdeg


def _combine2(self2, aggp, degp):
    B = 1280
    return pl.pallas_call(
        _combine2_block,
        grid=(NP // B,),
        in_specs=[
            pl.BlockSpec((B, NCLS), lambda i: (i, 0)),
            pl.BlockSpec((2, B, NCLS), lambda i: (0, i, 0)),
            pl.BlockSpec((2, B, 8), lambda i: (0, i, 0)),
        ],
        out_specs=pl.BlockSpec((B, NCLS), lambda i: (i, 0)),
        out_shape=jax.ShapeDtypeStruct((NP, NCLS), jnp.float32),
    )(self2, aggp, degp)


def kernel(features, edge_index, conv_w, conv_b, lin1_w, lin1_b, lin2_w,
           lin2_b, s1_self, s1_neigh, s1_b, s2_self, s2_neigh, s2_b):
    f32 = jnp.float32

    wp = jnp.arange(64)[:, None]
    ww = jnp.arange(64)[None, :]
    off = wp - ww + 1
    valid = (off >= 0) & (off <= 2)
    offc = jnp.clip(off, 0, 2)
    bds = []
    for dh in range(3):
        tap = conv_w[:, 0, dh, :]
        M = tap[:, offc]
        M = jnp.where(valid[None], M, 0.0)
        bds.append(jnp.transpose(M, (1, 0, 2)).reshape(64, 2048))
    bcat = jnp.concatenate(bds, axis=0)
    brow = jnp.repeat(conv_b, 64).reshape(1, 2048)

    cols = jnp.arange(2048)
    obase = (cols // 64) * 320 + (cols % 64) // 2
    even = (cols % 64) % 2 == 0
    w1p = jnp.stack([
        jnp.where(even[:, None], lin1_w[:, obase + ph * 32].T, 0.0)
        for ph in range(10)
    ])
    b1r = lin1_b.reshape(1, 32)
    w2p = lin2_w.T
    b2r = lin2_b.reshape(1, 32)

    feats = jnp.pad(features, ((0, NP - N), (0, 0), (0, 0)))
    feats2d = feats.reshape(NP * 20, 64)

    x = _cnn(feats2d, bcat, brow, w1p, b1r, w2p, b2r)

    pad = EP - E
    srcp = jnp.concatenate([edge_index[0],
                            jnp.zeros((pad,), jnp.int32)]).reshape(NW, KCH, CH)
    dstp = jnp.concatenate([edge_index[1],
                            jnp.full((pad,), NP, jnp.int32)]).reshape(NW, KCH, CH)

    z32 = jnp.zeros((NT, 32), f32)
    z16 = jnp.zeros((NT, 16), f32)
    z8 = jnp.zeros((NT, 8), f32)
    ones8 = jnp.ones((CH, 8), f32)

    agg1p, degp = _make_scatter(32, True)(x, srcp, dstp, z32, z8, ones8)

    p, self2 = _combine1(x, agg1p, degp, s1_self.T, s1_neigh.T,
                         s1_b.reshape(1, 64), s2_neigh.T, s2_self.T,
                         s2_b.reshape(1, NCLS))

    agg2p = _make_scatter(16, False)(p, srcp, dstp, z16)

    out = _combine2(self2, agg2p, degp)
    return out[:N]


# speedup vs baseline: 6.7981x; 2.0355x over previous
import functools

import jax
import jax.numpy as jnp
from jax import lax
from jax.experimental import pallas as pl
from jax.experimental.pallas import tpu as pltpu
from jax.experimental.pallas import tpu_sc as plsc

N = 10000
E = 640000
NCLS = 16

BN = 128
NP = 10240
NT = NP + 64
NW = 32
CH = 128
KCH = 160
EP = NW * KCH * CH


def _cnn_block(fe_ref, fo_ref, bcat_ref, brow_ref, w1_ref, b1_ref, w2_ref,
               b2_ref, out_ref):
    Fe = fe_ref[...]
    Fo = fo_ref[...]
    rows = BN * 10
    zpl = jnp.zeros((1, BN, 64), jnp.float32)
    fo_dn = jnp.concatenate([zpl, Fo[:-1]], axis=0)
    fe_up = jnp.concatenate([Fe[1:], zpl], axis=0)
    ge = jnp.concatenate([fo_dn, Fe, Fo],
                         axis=2).reshape(rows, 192).astype(jnp.bfloat16)
    go = jnp.concatenate([Fe, Fo, fe_up],
                         axis=2).reshape(rows, 192).astype(jnp.bfloat16)
    ye = jnp.dot(ge, bcat_ref[...], preferred_element_type=jnp.float32)
    yo = jnp.dot(go, bcat_ref[...], preferred_element_type=jnp.float32)
    yh = jnp.maximum(jnp.maximum(ye, yo) + brow_ref[...], 0.0)
    yw = jnp.maximum(yh, pltpu.roll(yh, shift=2047, axis=1))
    y3 = yw.reshape(10, BN, 2048)
    acc = jnp.broadcast_to(b1_ref[...], (BN, 32))
    for k in range(10):
        acc = acc + jnp.dot(y3[k], w1_ref[k],
                            preferred_element_type=jnp.float32)
    z = jnp.maximum(acc, 0.0)
    x = jnp.dot(z, w2_ref[...], preferred_element_type=jnp.float32)
    out_ref[...] = jnp.maximum(x + b2_ref[...], 0.0)


def _cnn(feats_e, feats_o, bcat, brow, w1p, b1r, w2p, b2r):
    return pl.pallas_call(
        _cnn_block,
        grid=(NP // BN,),
        in_specs=[
            pl.BlockSpec((10, BN, 64), lambda i: (0, i, 0)),
            pl.BlockSpec((10, BN, 64), lambda i: (0, i, 0)),
            pl.BlockSpec((192, 2048), lambda i: (0, 0)),
            pl.BlockSpec((1, 2048), lambda i: (0, 0)),
            pl.BlockSpec((10, 2048, 32), lambda i: (0, 0, 0)),
            pl.BlockSpec((1, 32), lambda i: (0, 0)),
            pl.BlockSpec((32, 32), lambda i: (0, 0)),
            pl.BlockSpec((1, 32), lambda i: (0, 0)),
        ],
        out_specs=pl.BlockSpec((BN, 32), lambda i: (i, 0)),
        out_shape=jax.ShapeDtypeStruct((NP, 32), jnp.float32),
    )(feats_e, feats_o, bcat, brow, w1p, b1r, w2p, b2r)


def _make_scatter(D, with_deg):
    mesh = plsc.VectorSubcoreMesh(core_axis_name="c", subcore_axis_name="s")
    out_type = [jax.ShapeDtypeStruct((2, NT, D), jnp.float32)]
    scratch = [
        pltpu.VMEM((KCH, CH), jnp.int32),
        pltpu.VMEM((KCH, CH), jnp.int32),
        pltpu.VMEM((CH, D), jnp.float32),
        pltpu.VMEM_SHARED((NT, D), jnp.float32),
        pltpu.SemaphoreType.DMA,
    ]
    if with_deg:
        out_type.append(jax.ShapeDtypeStruct((2, NT, 8), jnp.float32))
        scratch += [
            pltpu.VMEM((CH, 8), jnp.float32),
            pltpu.VMEM_SHARED((NT, 8), jnp.float32),
        ]

    def body(*refs):
        if with_deg:
            (x_hbm, src_hbm, dst_hbm, z_hbm, z8_hbm, ones_hbm,
             agg_out, deg_out, src_v, dst_v, msg_v, agg_sh, sem,
             ones_v, deg_sh) = refs
        else:
            (x_hbm, src_hbm, dst_hbm, z_hbm,
             agg_out, src_v, dst_v, msg_v, agg_sh, sem) = refs
        c = lax.axis_index("c")
        s = lax.axis_index("s")
        wid = s * 2 + c
        pltpu.sync_copy(src_hbm.at[wid], src_v)
        pltpu.sync_copy(dst_hbm.at[wid], dst_v)
        if with_deg:
            pltpu.sync_copy(ones_hbm, ones_v)

        @pl.when(s == 0)
        def _init():
            pltpu.sync_copy(z_hbm, agg_sh)
            if with_deg:
                pltpu.sync_copy(z8_hbm, deg_sh)

        plsc.subcore_barrier()

        def step(j, carry):
            pltpu.async_copy(x_hbm.at[src_v.at[j]], msg_v, sem).wait()
            pltpu.sync_copy(msg_v, agg_sh.at[dst_v.at[j]], add=True)
            if with_deg:
                pltpu.sync_copy(ones_v, deg_sh.at[dst_v.at[j]], add=True)
            return carry

        lax.fori_loop(0, KCH, step, 0)
        plsc.subcore_barrier()

        @pl.when(s == 0)
        def _flush():
            pltpu.sync_copy(agg_sh, agg_out.at[c])
            if with_deg:
                pltpu.sync_copy(deg_sh, deg_out.at[c])

    return pl.kernel(
        body,
        out_type=tuple(out_type) if with_deg else out_type[0],
        mesh=mesh,
        scratch_types=scratch,
        compiler_params=pltpu.CompilerParams(use_tc_tiling_on_sc=False),
    )


def _combine1_block(x_ref, a_ref, d_ref, s1s_ref, s1n_ref, s1b_ref,
                    s2n_ref, s2s_ref, s2b_ref, p_ref, self2_ref):
    agg = a_ref[0] + a_ref[1]
    deg = d_ref[0, :, 0:1] + d_ref[1, :, 0:1]
    rdeg = 1.0 / jnp.maximum(deg, 1.0)
    h = jnp.dot(x_ref[...], s1s_ref[...], preferred_element_type=jnp.float32)
    h = h + jnp.dot(agg * rdeg, s1n_ref[...],
                    preferred_element_type=jnp.float32)
    h = jnp.maximum(h + s1b_ref[...], 0.0)
    p_ref[...] = jnp.dot(h, s2n_ref[...], preferred_element_type=jnp.float32)
    self2_ref[...] = (jnp.dot(h, s2s_ref[...],
                              preferred_element_type=jnp.float32)
                      + s2b_ref[...])


def _combine1(x, aggp, degp, s1sT, s1nT, s1b, s2nT, s2sT, s2b):
    B = 1280
    return pl.pallas_call(
        _combine1_block,
        grid=(NP // B,),
        in_specs=[
            pl.BlockSpec((B, 32), lambda i: (i, 0)),
            pl.BlockSpec((2, B, 32), lambda i: (0, i, 0)),
            pl.BlockSpec((2, B, 8), lambda i: (0, i, 0)),
            pl.BlockSpec((32, 64), lambda i: (0, 0)),
            pl.BlockSpec((32, 64), lambda i: (0, 0)),
            pl.BlockSpec((1, 64), lambda i: (0, 0)),
            pl.BlockSpec((64, NCLS), lambda i: (0, 0)),
            pl.BlockSpec((64, NCLS), lambda i: (0, 0)),
            pl.BlockSpec((1, NCLS), lambda i: (0, 0)),
        ],
        out_specs=[
            pl.BlockSpec((B, NCLS), lambda i: (i, 0)),
            pl.BlockSpec((B, NCLS), lambda i: (i, 0)),
        ],
        out_shape=[
            jax.ShapeDtypeStruct((NP, NCLS), jnp.float32),
            jax.ShapeDtypeStruct((NP, NCLS), jnp.float32),
        ],
    )(x, aggp, degp, s1sT, s1nT, s1b, s2nT, s2sT, s2b)


def _combine2_block(self2_ref, a_ref, d_ref, out_ref):
    agg = a_ref[0] + a_ref[1]
    deg = d_ref[0, :, 0:1] + d_ref[1, :, 0:1]
    rdeg = 1.0 / jnp.maximum(deg, 1.0)
    out_ref[...] = self2_ref[...] + agg * rdeg


def _combine2(self2, aggp, degp):
    B = 1280
    return pl.pallas_call(
        _combine2_block,
        grid=(NP // B,),
        in_specs=[
            pl.BlockSpec((B, NCLS), lambda i: (i, 0)),
            pl.BlockSpec((2, B, NCLS), lambda i: (0, i, 0)),
            pl.BlockSpec((2, B, 8), lambda i: (0, i, 0)),
        ],
        out_specs=pl.BlockSpec((B, NCLS), lambda i: (i, 0)),
        out_shape=jax.ShapeDtypeStruct((NP, NCLS), jnp.float32),
    )(self2, aggp, degp)


def kernel(features, edge_index, conv_w, conv_b, lin1_w, lin1_b, lin2_w,
           lin2_b, s1_self, s1_neigh, s1_b, s2_self, s2_neigh, s2_b):
    f32 = jnp.float32

    wp = jnp.arange(64)[:, None]
    ww = jnp.arange(64)[None, :]
    off = wp - ww + 1
    valid = (off >= 0) & (off <= 2)
    offc = jnp.clip(off, 0, 2)
    bds = []
    for dh in range(3):
        tap = conv_w[:, 0, dh, :]
        M = tap[:, offc]
        M = jnp.where(valid[None], M, 0.0)
        bds.append(jnp.transpose(M, (1, 0, 2)).reshape(64, 2048))
    bcat = jnp.concatenate(bds, axis=0)
    brow = jnp.repeat(conv_b, 64).reshape(1, 2048)

    cols = jnp.arange(2048)
    obase = (cols // 64) * 320 + (cols % 64) // 2
    even = (cols % 64) % 2 == 0
    w1p = jnp.stack([
        jnp.where(even[:, None], lin1_w[:, obase + ph * 32].T, 0.0)
        for ph in range(10)
    ])
    b1r = lin1_b.reshape(1, 32)
    w2p = lin2_w.T
    b2r = lin2_b.reshape(1, 32)

    feats = jnp.pad(features, ((0, NP - N), (0, 0), (0, 0)))
    feats_e = feats[:, 0::2, :].transpose(1, 0, 2)
    feats_o = feats[:, 1::2, :].transpose(1, 0, 2)

    x = _cnn(feats_e, feats_o, bcat.astype(jnp.bfloat16), brow, w1p, b1r,
             w2p, b2r)

    pad = EP - E
    srcp = jnp.concatenate([edge_index[0],
                            jnp.zeros((pad,), jnp.int32)]).reshape(NW, KCH, CH)
    dstp = jnp.concatenate([edge_index[1],
                            jnp.full((pad,), NP, jnp.int32)]).reshape(NW, KCH, CH)

    z32 = jnp.zeros((NT, 32), f32)
    z16 = jnp.zeros((NT, 16), f32)
    z8 = jnp.zeros((NT, 8), f32)
    ones8 = jnp.ones((CH, 8), f32)

    agg1p, degp = _make_scatter(32, True)(x, srcp, dstp, z32, z8, ones8)

    p, self2 = _combine1(x, agg1p, degp, s1_self.T, s1_neigh.T,
                         s1_b.reshape(1, 64), s2_neigh.T, s2_self.T,
                         s2_b.reshape(1, NCLS))

    agg2p = _make_scatter(16, False)(p, srcp, dstp, z16)

    out = _combine2(self2, agg2p, degp)
    return out[:N]


# speedup vs baseline: 7.9217x; 1.1653x over previous
import functools

import jax
import jax.numpy as jnp
from jax import lax
from jax.experimental import pallas as pl
from jax.experimental.pallas import tpu as pltpu
from jax.experimental.pallas import tpu_sc as plsc

N = 10000
E = 640000
NCLS = 16

BN = 128
NP = 10240
NT = NP + 64
NW = 32
CH = 128
KCH = 160
EP = NW * KCH * CH


def _cnn_block(fe_ref, fo_ref, bcat_ref, brow_ref, w1_ref, b1_ref, w2_ref,
               b2_ref, out_ref):
    Fe = fe_ref[...]
    Fo = fo_ref[...]
    rows = BN * 10
    zpl = jnp.zeros((1, BN, 64), jnp.float32)
    fo_dn = jnp.concatenate([zpl, Fo[:-1]], axis=0)
    fe_up = jnp.concatenate([Fe[1:], zpl], axis=0)
    ge = jnp.concatenate([fo_dn, Fe, Fo],
                         axis=2).reshape(rows, 192).astype(jnp.bfloat16)
    go = jnp.concatenate([Fe, Fo, fe_up],
                         axis=2).reshape(rows, 192).astype(jnp.bfloat16)
    ye = jnp.dot(ge, bcat_ref[...], preferred_element_type=jnp.float32)
    yo = jnp.dot(go, bcat_ref[...], preferred_element_type=jnp.float32)
    yh = jnp.maximum(jnp.maximum(ye, yo) + brow_ref[...], 0.0)
    yw = jnp.maximum(yh, pltpu.roll(yh, shift=2047, axis=1))
    y3 = yw.reshape(10, BN, 2048)
    acc = jnp.broadcast_to(b1_ref[...], (BN, 32))
    for k in range(10):
        acc = acc + jnp.dot(y3[k], w1_ref[k],
                            preferred_element_type=jnp.float32)
    z = jnp.maximum(acc, 0.0)
    x = jnp.dot(z, w2_ref[...], preferred_element_type=jnp.float32)
    out_ref[...] = jnp.maximum(x + b2_ref[...], 0.0)


def _cnn(feats_e, feats_o, bcat, brow, w1p, b1r, w2p, b2r):
    return pl.pallas_call(
        _cnn_block,
        grid=(NP // BN,),
        in_specs=[
            pl.BlockSpec((10, BN, 64), lambda i: (0, i, 0)),
            pl.BlockSpec((10, BN, 64), lambda i: (0, i, 0)),
            pl.BlockSpec((192, 2048), lambda i: (0, 0)),
            pl.BlockSpec((1, 2048), lambda i: (0, 0)),
            pl.BlockSpec((10, 2048, 32), lambda i: (0, 0, 0)),
            pl.BlockSpec((1, 32), lambda i: (0, 0)),
            pl.BlockSpec((32, 32), lambda i: (0, 0)),
            pl.BlockSpec((1, 32), lambda i: (0, 0)),
        ],
        out_specs=pl.BlockSpec((BN, 32), lambda i: (i, 0)),
        out_shape=jax.ShapeDtypeStruct((NP, 32), jnp.float32),
    )(feats_e, feats_o, bcat, brow, w1p, b1r, w2p, b2r)


def _make_scatter(D, with_deg):
    mesh = plsc.VectorSubcoreMesh(core_axis_name="c", subcore_axis_name="s")
    out_type = [jax.ShapeDtypeStruct((2, NT, D), jnp.float32)]
    NB = 4
    scratch = [
        pltpu.VMEM((KCH, CH), jnp.int32),
        pltpu.VMEM((KCH, CH), jnp.int32),
    ]
    scratch += [pltpu.VMEM((CH, D), jnp.float32) for _ in range(NB)]
    scratch += [pltpu.VMEM_SHARED((NT, D), jnp.float32)]
    scratch += [pltpu.SemaphoreType.DMA for _ in range(2 * NB)]
    if with_deg:
        out_type.append(jax.ShapeDtypeStruct((2, NT, 8), jnp.float32))
        scratch += [
            pltpu.VMEM((CH, 8), jnp.float32),
            pltpu.VMEM_SHARED((NT, 8), jnp.float32),
        ]
        scratch += [pltpu.SemaphoreType.DMA for _ in range(NB)]

    def body(*refs):
        if with_deg:
            (x_hbm, src_hbm, dst_hbm, z_hbm, z8_hbm, ones_hbm,
             agg_out, deg_out, src_v, dst_v, m0, m1, m2, m3, agg_sh,
             g0, g1, g2, g3, s0, s1, s2, s3,
             ones_v, deg_sh, d0, d1, d2, d3) = refs
            dsem = [d0, d1, d2, d3]
        else:
            (x_hbm, src_hbm, dst_hbm, z_hbm,
             agg_out, src_v, dst_v, m0, m1, m2, m3, agg_sh,
             g0, g1, g2, g3, s0, s1, s2, s3) = refs
        msg = [m0, m1, m2, m3]
        gsem = [g0, g1, g2, g3]
        ssem = [s0, s1, s2, s3]
        c = lax.axis_index("c")
        s = lax.axis_index("s")
        wid = s * 2 + c
        pltpu.sync_copy(src_hbm.at[wid], src_v)
        pltpu.sync_copy(dst_hbm.at[wid], dst_v)
        if with_deg:
            pltpu.sync_copy(ones_hbm, ones_v)

        @pl.when(s == 0)
        def _init():
            pltpu.sync_copy(z_hbm, agg_sh)
            if with_deg:
                pltpu.sync_copy(z8_hbm, deg_sh)

        plsc.subcore_barrier()

        pltpu.async_copy(x_hbm.at[src_v.at[0]], msg[0], gsem[0])
        pltpu.async_copy(x_hbm.at[src_v.at[1]], msg[1], gsem[1])

        def do_chunk(t, b):
            b2 = (b + 2) % NB
            pltpu.make_async_copy(x_hbm.at[src_v.at[t]], msg[b],
                                  gsem[b]).wait()
            pltpu.async_copy(msg[b], agg_sh.at[dst_v.at[t]], ssem[b],
                             add=True)
            if with_deg:
                pltpu.async_copy(ones_v, deg_sh.at[dst_v.at[t]], dsem[b],
                                 add=True)

            @pl.when(t >= 2)
            def _wait_prev():
                pltpu.make_async_copy(msg[b2], agg_sh.at[dst_v.at[t]],
                                      ssem[b2]).wait()
                if with_deg:
                    pltpu.make_async_copy(ones_v, deg_sh.at[dst_v.at[t]],
                                          dsem[b2]).wait()

            @pl.when(t + 2 < KCH)
            def _next_gather():
                pltpu.async_copy(x_hbm.at[src_v.at[t + 2]], msg[b2],
                                 gsem[b2])

        def step(tt, carry):
            for b in range(NB):
                do_chunk(tt * NB + b, b)
            return carry

        lax.fori_loop(0, KCH // NB, step, 0)
        for b in (2, 3):
            pltpu.make_async_copy(msg[b], agg_sh.at[dst_v.at[0]],
                                  ssem[b]).wait()
            if with_deg:
                pltpu.make_async_copy(ones_v, deg_sh.at[dst_v.at[0]],
                                      dsem[b]).wait()
        plsc.subcore_barrier()

        @pl.when(s == 0)
        def _flush():
            pltpu.sync_copy(agg_sh, agg_out.at[c])
            if with_deg:
                pltpu.sync_copy(deg_sh, deg_out.at[c])

    return pl.kernel(
        body,
        out_type=tuple(out_type) if with_deg else out_type[0],
        mesh=mesh,
        scratch_types=scratch,
        compiler_params=pltpu.CompilerParams(use_tc_tiling_on_sc=False),
    )


def _combine1_block(x_ref, a_ref, d_ref, s1s_ref, s1n_ref, s1b_ref,
                    s2n_ref, s2s_ref, s2b_ref, p_ref, self2_ref):
    agg = a_ref[0] + a_ref[1]
    deg = d_ref[0, :, 0:1] + d_ref[1, :, 0:1]
    rdeg = 1.0 / jnp.maximum(deg, 1.0)
    h = jnp.dot(x_ref[...], s1s_ref[...], preferred_element_type=jnp.float32)
    h = h + jnp.dot(agg * rdeg, s1n_ref[...],
                    preferred_element_type=jnp.float32)
    h = jnp.maximum(h + s1b_ref[...], 0.0)
    p_ref[...] = jnp.dot(h, s2n_ref[...], preferred_element_type=jnp.float32)
    self2_ref[...] = (jnp.dot(h, s2s_ref[...],
                              preferred_element_type=jnp.float32)
                      + s2b_ref[...])


def _combine1(x, aggp, degp, s1sT, s1nT, s1b, s2nT, s2sT, s2b):
    B = 1280
    return pl.pallas_call(
        _combine1_block,
        grid=(NP // B,),
        in_specs=[
            pl.BlockSpec((B, 32), lambda i: (i, 0)),
            pl.BlockSpec((2, B, 32), lambda i: (0, i, 0)),
            pl.BlockSpec((2, B, 8), lambda i: (0, i, 0)),
            pl.BlockSpec((32, 64), lambda i: (0, 0)),
            pl.BlockSpec((32, 64), lambda i: (0, 0)),
            pl.BlockSpec((1, 64), lambda i: (0, 0)),
            pl.BlockSpec((64, NCLS), lambda i: (0, 0)),
            pl.BlockSpec((64, NCLS), lambda i: (0, 0)),
            pl.BlockSpec((1, NCLS), lambda i: (0, 0)),
        ],
        out_specs=[
            pl.BlockSpec((B, NCLS), lambda i: (i, 0)),
            pl.BlockSpec((B, NCLS), lambda i: (i, 0)),
        ],
        out_shape=[
            jax.ShapeDtypeStruct((NP, NCLS), jnp.float32),
            jax.ShapeDtypeStruct((NP, NCLS), jnp.float32),
        ],
    )(x, aggp, degp, s1sT, s1nT, s1b, s2nT, s2sT, s2b)


def _combine2_block(self2_ref, a_ref, d_ref, out_ref):
    agg = a_ref[0] + a_ref[1]
    deg = d_ref[0, :, 0:1] + d_ref[1, :, 0:1]
    rdeg = 1.0 / jnp.maximum(deg, 1.0)
    out_ref[...] = self2_ref[...] + agg * rdeg


def _combine2(self2, aggp, degp):
    B = 1280
    return pl.pallas_call(
        _combine2_block,
        grid=(NP // B,),
        in_specs=[
            pl.BlockSpec((B, NCLS), lambda i: (i, 0)),
            pl.BlockSpec((2, B, NCLS), lambda i: (0, i, 0)),
            pl.BlockSpec((2, B, 8), lambda i: (0, i, 0)),
        ],
        out_specs=pl.BlockSpec((B, NCLS), lambda i: (i, 0)),
        out_shape=jax.ShapeDtypeStruct((NP, NCLS), jnp.float32),
    )(self2, aggp, degp)


def kernel(features, edge_index, conv_w, conv_b, lin1_w, lin1_b, lin2_w,
           lin2_b, s1_self, s1_neigh, s1_b, s2_self, s2_neigh, s2_b):
    f32 = jnp.float32

    wp = jnp.arange(64)[:, None]
    ww = jnp.arange(64)[None, :]
    off = wp - ww + 1
    valid = (off >= 0) & (off <= 2)
    offc = jnp.clip(off, 0, 2)
    bds = []
    for dh in range(3):
        tap = conv_w[:, 0, dh, :]
        M = tap[:, offc]
        M = jnp.where(valid[None], M, 0.0)
        bds.append(jnp.transpose(M, (1, 0, 2)).reshape(64, 2048))
    bcat = jnp.concatenate(bds, axis=0)
    brow = jnp.repeat(conv_b, 64).reshape(1, 2048)

    cols = jnp.arange(2048)
    obase = (cols // 64) * 320 + (cols % 64) // 2
    even = (cols % 64) % 2 == 0
    w1p = jnp.stack([
        jnp.where(even[:, None], lin1_w[:, obase + ph * 32].T, 0.0)
        for ph in range(10)
    ])
    b1r = lin1_b.reshape(1, 32)
    w2p = lin2_w.T
    b2r = lin2_b.reshape(1, 32)

    feats = jnp.pad(features, ((0, NP - N), (0, 0), (0, 0)))
    feats_e = feats[:, 0::2, :].transpose(1, 0, 2)
    feats_o = feats[:, 1::2, :].transpose(1, 0, 2)

    x = _cnn(feats_e, feats_o, bcat.astype(jnp.bfloat16), brow, w1p, b1r,
             w2p, b2r)

    pad = EP - E
    srcp = jnp.concatenate([edge_index[0],
                            jnp.zeros((pad,), jnp.int32)]).reshape(NW, KCH, CH)
    dstp = jnp.concatenate([edge_index[1],
                            jnp.full((pad,), NP, jnp.int32)]).reshape(NW, KCH, CH)

    z32 = jnp.zeros((NT, 32), f32)
    z16 = jnp.zeros((NT, 16), f32)
    z8 = jnp.zeros((NT, 8), f32)
    ones8 = jnp.ones((CH, 8), f32)

    agg1p, degp = _make_scatter(32, True)(x, srcp, dstp, z32, z8, ones8)

    p, self2 = _combine1(x, agg1p, degp, s1_self.T, s1_neigh.T,
                         s1_b.reshape(1, 64), s2_neigh.T, s2_self.T,
                         s2_b.reshape(1, NCLS))

    agg2p = _make_scatter(16, False)(p, srcp, dstp, z16)

    out = _combine2(self2, agg2p, degp)
    return out[:N]
